# Initial kernel scaffold; baseline (speedup 1.0000x reference)
import jax
import jax.numpy as jnp
from jax.experimental import pallas as pl


def kernel(x, emb, Wf_ih, Wf_hh, bf_ih, bf_hh, Wb_ih, Wb_hh, bb_ih, bb_hh, W1, b1, W2, b2):
    raise NotImplementedError("write your pallas kernel here")


# speedup vs baseline: 5.4043x; 5.4043x over previous
import functools

import jax
import jax.numpy as jnp
from jax import lax
from jax.experimental import pallas as pl
from jax.experimental.pallas import tpu as pltpu
from jax.experimental.pallas import tpu_sc as plsc

VOCAB_ = 20000
EMBED_ = 128
HID_ = 50
B_ = 1024
L_ = 100

_NC, _NS = 2, 16
_NW = _NC * _NS
_NIDX = B_ * L_
_BPW = _NIDX // _NW
_CHUNK = 320
_NCHUNK = _BPW // _CHUNK


def _sc_gather(emb, idx):
    mesh = plsc.VectorSubcoreMesh(
        core_axis_name="c", subcore_axis_name="s",
        num_cores=_NC, num_subcores=_NS)

    @functools.partial(
        pl.kernel,
        out_type=jax.ShapeDtypeStruct((_NIDX, EMBED_), jnp.float32),
        mesh=mesh,
        scratch_types=[
            pltpu.VMEM((_BPW,), jnp.int32),
            pltpu.VMEM((_CHUNK, EMBED_), jnp.float32),
            pltpu.VMEM((_CHUNK, EMBED_), jnp.float32),
            pltpu.SemaphoreType.DMA,
            pltpu.SemaphoreType.DMA,
        ],
    )
    def gather_k(table_hbm, idx_hbm, out_hbm, idx_v, buf0, buf1, sem0, sem1):
        wid = lax.axis_index("s") * _NC + lax.axis_index("c")
        base = wid * _BPW
        pltpu.sync_copy(idx_hbm.at[pl.ds(base, _BPW)], idx_v)
        bufs = (buf0, buf1)
        sems = (sem0, sem1)
        copies = [None, None]
        copies[0] = pltpu.async_copy(
            table_hbm.at[idx_v.at[pl.ds(0, _CHUNK)]], buf0, sem0)
        for c in range(_NCHUNK):
            if c + 1 < _NCHUNK:
                copies[(c + 1) % 2] = pltpu.async_copy(
                    table_hbm.at[idx_v.at[pl.ds((c + 1) * _CHUNK, _CHUNK)]],
                    bufs[(c + 1) % 2], sems[(c + 1) % 2])
            copies[c % 2].wait()
            pltpu.sync_copy(bufs[c % 2],
                            out_hbm.at[pl.ds(base + c * _CHUNK, _CHUNK)])

    return gather_k(emb, idx)


def _perm_gates(w):
    return jnp.concatenate([w[0:2 * HID_], w[3 * HID_:4 * HID_],
                            w[2 * HID_:3 * HID_]], axis=0)


def _build_big(w_ih, w_hh, bias):
    return jnp.concatenate([
        _perm_gates(w_ih).T,
        _perm_gates(w_hh).T,
        _perm_gates(bias[:, None]).T,
    ], axis=0)


def _lstm_body(ef, eb, wf, wb, mf, mb, af, ab, cf, cb):
    t = pl.program_id(0)

    @pl.when(t == 0)
    def _init():
        zeros = jnp.zeros((B_, HID_), jnp.float32)
        ones = jnp.ones((B_, 1), jnp.float32)
        af[:, EMBED_:EMBED_ + HID_] = zeros
        ab[:, EMBED_:EMBED_ + HID_] = zeros
        af[:, EMBED_ + HID_:] = ones
        ab[:, EMBED_ + HID_:] = ones
        cf[...] = zeros
        cb[...] = zeros

    for e_ref, w_ref, m_ref, a_ref, c_ref in (
            (ef, wf, mf, af, cf), (eb, wb, mb, ab, cb)):
        a_ref[:, 0:EMBED_] = e_ref[0]
        g = jnp.dot(a_ref[...], w_ref[...],
                    preferred_element_type=jnp.float32)
        s = jax.nn.sigmoid(g[:, 0:3 * HID_])
        gg = jnp.tanh(g[:, 3 * HID_:4 * HID_])
        c_new = s[:, HID_:2 * HID_] * c_ref[...] + s[:, 0:HID_] * gg
        h_new = s[:, 2 * HID_:3 * HID_] * jnp.tanh(c_new)
        c_ref[...] = c_new
        a_ref[:, EMBED_:EMBED_ + HID_] = h_new
        m_ref[0] = jnp.max(h_new, axis=1, keepdims=True)


def _lstm(e_tm, wf, wb):
    kdim = EMBED_ + HID_ + 1
    return pl.pallas_call(
        _lstm_body,
        grid=(L_,),
        in_specs=[
            pl.BlockSpec((1, B_, EMBED_), lambda t: (t, 0, 0)),
            pl.BlockSpec((1, B_, EMBED_), lambda t: (L_ - 1 - t, 0, 0)),
            pl.BlockSpec((kdim, 4 * HID_), lambda t: (0, 0)),
            pl.BlockSpec((kdim, 4 * HID_), lambda t: (0, 0)),
        ],
        out_specs=[
            pl.BlockSpec((1, B_, 1), lambda t: (t, 0, 0)),
            pl.BlockSpec((1, B_, 1), lambda t: (L_ - 1 - t, 0, 0)),
        ],
        out_shape=[
            jax.ShapeDtypeStruct((L_, B_, 1), jnp.float32),
            jax.ShapeDtypeStruct((L_, B_, 1), jnp.float32),
        ],
        scratch_shapes=[
            pltpu.VMEM((B_, kdim), jnp.float32),
            pltpu.VMEM((B_, kdim), jnp.float32),
            pltpu.VMEM((B_, HID_), jnp.float32),
            pltpu.VMEM((B_, HID_), jnp.float32),
        ],
        compiler_params=pltpu.CompilerParams(
            dimension_semantics=("arbitrary",)),
    )(e_tm, e_tm, wf, wb)


def _mlp_body(mf, mb, w1, b1, w2, b2, out):
    a = jnp.maximum(mf[...], mb[...])
    z = jnp.dot(w1[...], a, preferred_element_type=jnp.float32) + b1[...]
    z = jnp.maximum(z, 0.0)
    o = jnp.dot(w2[...], z, preferred_element_type=jnp.float32) + b2[...]
    out[...] = jax.nn.sigmoid(o)


def _mlp(mf, mb, w1, b1, w2, b2):
    return pl.pallas_call(
        _mlp_body,
        out_shape=jax.ShapeDtypeStruct((w2.shape[0], B_), jnp.float32),
    )(mf, mb, w1, b1, w2, b2)


def kernel(x, emb, Wf_ih, Wf_hh, bf_ih, bf_hh, Wb_ih, Wb_hh, bb_ih, bb_hh,
           W1, b1, W2, b2):
    xt = x.T.reshape(-1).astype(jnp.int32)
    e_flat = _sc_gather(emb, xt)
    e_tm = e_flat.reshape(L_, B_, EMBED_)
    wf = _build_big(Wf_ih, Wf_hh, bf_ih + bf_hh)
    wb = _build_big(Wb_ih, Wb_hh, bb_ih + bb_hh)
    mf, mb = _lstm(e_tm, wf, wb)
    out6 = _mlp(mf.reshape(L_, B_), mb.reshape(L_, B_),
                W1, b1.reshape(-1, 1), W2, b2.reshape(-1, 1))
    return out6.T


# speedup vs baseline: 8.1457x; 1.5073x over previous
import functools

import jax
import jax.numpy as jnp
from jax import lax
from jax.experimental import pallas as pl
from jax.experimental.pallas import tpu as pltpu
from jax.experimental.pallas import tpu_sc as plsc

VOCAB_ = 20000
EMBED_ = 128
HID_ = 50
B_ = 1024
L_ = 100

_NC, _NS = 2, 16
_NW = _NC * _NS
_NIDX = B_ * L_
_BPW = _NIDX // _NW
_CHUNK = 320
_NCHUNK = _BPW // _CHUNK


def _sc_gather(emb, idx):
    mesh = plsc.VectorSubcoreMesh(
        core_axis_name="c", subcore_axis_name="s",
        num_cores=_NC, num_subcores=_NS)

    @functools.partial(
        pl.kernel,
        out_type=jax.ShapeDtypeStruct((_NIDX, EMBED_), jnp.float32),
        mesh=mesh,
        scratch_types=[
            pltpu.VMEM((_BPW,), jnp.int32),
            pltpu.VMEM((_CHUNK, EMBED_), jnp.float32),
            pltpu.VMEM((_CHUNK, EMBED_), jnp.float32),
            pltpu.SemaphoreType.DMA,
            pltpu.SemaphoreType.DMA,
        ],
    )
    def gather_k(table_hbm, idx_hbm, out_hbm, idx_v, buf0, buf1, sem0, sem1):
        wid = lax.axis_index("s") * _NC + lax.axis_index("c")
        base = wid * _BPW
        pltpu.sync_copy(idx_hbm.at[pl.ds(base, _BPW)], idx_v)
        bufs = (buf0, buf1)
        sems = (sem0, sem1)
        copies = [None, None]
        copies[0] = pltpu.async_copy(
            table_hbm.at[idx_v.at[pl.ds(0, _CHUNK)]], buf0, sem0)
        for c in range(_NCHUNK):
            if c + 1 < _NCHUNK:
                copies[(c + 1) % 2] = pltpu.async_copy(
                    table_hbm.at[idx_v.at[pl.ds((c + 1) * _CHUNK, _CHUNK)]],
                    bufs[(c + 1) % 2], sems[(c + 1) % 2])
            copies[c % 2].wait()
            pltpu.sync_copy(bufs[c % 2],
                            out_hbm.at[pl.ds(base + c * _CHUNK, _CHUNK)])

    return gather_k(emb, idx)


_GPAD = 128


def _perm_gates(w):
    return jnp.concatenate([w[0:2 * HID_], w[3 * HID_:4 * HID_],
                            w[2 * HID_:3 * HID_]], axis=0)


def _build_big(w_ih, w_hh, bias):
    w200 = jnp.concatenate([
        _perm_gates(w_ih).T,
        _perm_gates(w_hh).T,
        _perm_gates(bias[:, None]).T,
    ], axis=0)
    w = jnp.zeros((w200.shape[0], 4 * _GPAD), w200.dtype)
    for q in range(4):
        w = w.at[:, q * _GPAD:q * _GPAD + HID_].set(
            w200[:, q * HID_:(q + 1) * HID_])
    return w


def _lstm_body(ef, eb, wf, wb, mf, mb, af, ab, cf, cb):
    t = pl.program_id(0)

    @pl.when(t == 0)
    def _init():
        zeros = jnp.zeros((B_, HID_), jnp.float32)
        ones = jnp.ones((B_, 1), jnp.float32)
        af[:, EMBED_:EMBED_ + HID_] = zeros
        ab[:, EMBED_:EMBED_ + HID_] = zeros
        af[:, EMBED_ + HID_:] = ones
        ab[:, EMBED_ + HID_:] = ones
        cf[...] = zeros
        cb[...] = zeros

    for e_ref, w_ref, m_ref, a_ref, c_ref in (
            (ef, wf, mf, af, cf), (eb, wb, mb, ab, cb)):
        a_ref[:, 0:EMBED_] = e_ref[0]
        g = jnp.dot(a_ref[...], w_ref[...],
                    preferred_element_type=jnp.float32)
        i = 0.5 * jnp.tanh(0.5 * g[:, 0:HID_]) + 0.5
        f = 0.5 * jnp.tanh(0.5 * g[:, _GPAD:_GPAD + HID_]) + 0.5
        o = 0.5 * jnp.tanh(0.5 * g[:, 2 * _GPAD:2 * _GPAD + HID_]) + 0.5
        gg = jnp.tanh(g[:, 3 * _GPAD:3 * _GPAD + HID_])
        c_new = f * c_ref[...] + i * gg
        h_new = o * jnp.tanh(c_new)
        c_ref[...] = c_new
        a_ref[:, EMBED_:EMBED_ + HID_] = h_new
        m_ref[0] = jnp.max(h_new, axis=1, keepdims=True)


def _lstm(e_tm, wf, wb):
    kdim = EMBED_ + HID_ + 1
    return pl.pallas_call(
        _lstm_body,
        grid=(L_,),
        in_specs=[
            pl.BlockSpec((1, B_, EMBED_), lambda t: (t, 0, 0)),
            pl.BlockSpec((1, B_, EMBED_), lambda t: (L_ - 1 - t, 0, 0)),
            pl.BlockSpec((kdim, 4 * _GPAD), lambda t: (0, 0)),
            pl.BlockSpec((kdim, 4 * _GPAD), lambda t: (0, 0)),
        ],
        out_specs=[
            pl.BlockSpec((1, B_, 1), lambda t: (t, 0, 0)),
            pl.BlockSpec((1, B_, 1), lambda t: (L_ - 1 - t, 0, 0)),
        ],
        out_shape=[
            jax.ShapeDtypeStruct((L_, B_, 1), jnp.float32),
            jax.ShapeDtypeStruct((L_, B_, 1), jnp.float32),
        ],
        scratch_shapes=[
            pltpu.VMEM((B_, kdim), jnp.float32),
            pltpu.VMEM((B_, kdim), jnp.float32),
            pltpu.VMEM((B_, HID_), jnp.float32),
            pltpu.VMEM((B_, HID_), jnp.float32),
        ],
        compiler_params=pltpu.CompilerParams(
            dimension_semantics=("arbitrary",)),
    )(e_tm, e_tm, wf, wb)


def _mlp_body(mf, mb, w1, b1, w2, b2, out):
    a = jnp.maximum(mf[...], mb[...])
    z = jnp.dot(w1[...], a, preferred_element_type=jnp.float32) + b1[...]
    z = jnp.maximum(z, 0.0)
    o = jnp.dot(w2[...], z, preferred_element_type=jnp.float32) + b2[...]
    out[...] = jax.nn.sigmoid(o)


def _mlp(mf, mb, w1, b1, w2, b2):
    return pl.pallas_call(
        _mlp_body,
        out_shape=jax.ShapeDtypeStruct((w2.shape[0], B_), jnp.float32),
    )(mf, mb, w1, b1, w2, b2)


def kernel(x, emb, Wf_ih, Wf_hh, bf_ih, bf_hh, Wb_ih, Wb_hh, bb_ih, bb_hh,
           W1, b1, W2, b2):
    xt = x.T.reshape(-1).astype(jnp.int32)
    e_flat = _sc_gather(emb, xt)
    e_tm = e_flat.reshape(L_, B_, EMBED_)
    wf = _build_big(Wf_ih, Wf_hh, bf_ih + bf_hh)
    wb = _build_big(Wb_ih, Wb_hh, bb_ih + bb_hh)
    mf, mb = _lstm(e_tm, wf, wb)
    out6 = _mlp(mf.reshape(L_, B_), mb.reshape(L_, B_),
                W1, b1.reshape(-1, 1), W2, b2.reshape(-1, 1))
    return out6.T
